# baseline (speedup 1.0000x reference)
import functools

import jax
import jax.numpy as jnp
import numpy as np
from jax import lax
from jax.experimental import pallas as pl
from jax.experimental.pallas import tpu as pltpu
from jax.experimental.pallas import tpu_sc as plsc

EMBED = 256
HEADS = 8
HD = EMBED // HEADS
HALF = HD // 2
NLEV = 4
NH_SIZES = [3, 5, 7, 9]
NKEY = sum(s * s for s in NH_SIZES)
KPAD = 192
MAXHW = 96

_PERM = np.concatenate([np.arange(0, EMBED, 2), np.arange(1, EMBED, 2)])
_HI_MASK = np.int32(np.uint32(0xFFFF0000).view(np.int32))


def _pack_bf16(x1, x2):
    b1 = lax.bitcast_convert_type(x1, jnp.int32)
    b2 = lax.bitcast_convert_type(x2, jnp.int32)
    r1 = b1 + 0x7FFF + (lax.shift_right_logical(b1, 16) & 1)
    r2 = b2 + 0x7FFF + (lax.shift_right_logical(b2, 16) & 1)
    return (r1 & _HI_MASK) | lax.shift_right_logical(r2, 16)


def _unpack_hi(p):
    return lax.bitcast_convert_type(p & _HI_MASK, jnp.float32)


def _unpack_lo(p):
    return lax.bitcast_convert_type(p << 16, jnp.float32)


def _build_offset_grids():
    grids = []
    for s in NH_SIZES:
        ax = np.arange(s)
        g = np.stack(np.meshgrid(ax, ax, indexing='ij'), -1).reshape(-1, 2) - (s - 1) // 2
        grids.append(g.astype(np.int32))
    lev = np.concatenate([np.full(s * s, l, np.int32) for l, s in enumerate(NH_SIZES)])
    return grids, lev


def _q_body(q_ref, pos_ref, w_ref, nw_ref, nb_ref, f3_ref, lv_ref, out_ref):
    q = q_ref[...]
    mu = jnp.mean(q, axis=-1, keepdims=True)
    var = jnp.mean((q - mu) ** 2, axis=-1, keepdims=True)
    qn = (q - mu) * jax.lax.rsqrt(var + 1e-5) * nw_ref[...] + nb_ref[...]
    qp = jnp.dot(qn, w_ref[...], preferred_element_type=jnp.float32)
    ang = (pos_ref[:, 0:1] * f3_ref[0:1, :] + pos_ref[:, 1:2] * f3_ref[1:2, :]
           + lv_ref[...])
    c = jnp.cos(ang)
    s = jnp.sin(ang)
    x1 = qp[:, :128]
    x2 = qp[:, 128:]
    scale = 1.0 / np.sqrt(np.float32(HD))
    out_ref[...] = jnp.concatenate([(x1 * c - x2 * s) * scale,
                                    (x1 * s + x2 * c) * scale], axis=-1)


def _kv_body(s_ref, wk_ref, wv_ref, f3_ref, sb_ref, kv_ref, *, blk):
    i = pl.program_id(0)
    feats = s_ref[...]
    r = i * blk + lax.broadcasted_iota(jnp.int32, (blk, 1), 0)
    l_r = r % NLEV
    q4 = r // NLEV
    x_r = q4 % MAXHW
    y_r = (q4 // MAXHW) % MAXHW
    lf = l_r.astype(jnp.float32)
    sby = jnp.zeros((blk, 1), jnp.float32)
    sbx = jnp.zeros((blk, 1), jnp.float32)
    for l in range(NLEV):
        m = (l_r == l).astype(jnp.float32)
        sby = sby + m * sb_ref[0:1, l:l + 1]
        sbx = sbx + m * sb_ref[0:1, NLEV + l:NLEV + l + 1]
    py = (y_r.astype(jnp.float32) + 0.5) * sby - 0.5
    px = (x_r.astype(jnp.float32) + 0.5) * sbx - 0.5
    ang = py * f3_ref[0:1, :] + px * f3_ref[1:2, :] + lf * f3_ref[2:3, :]
    c = jnp.cos(ang)
    s = jnp.sin(ang)
    kp = jnp.dot(feats, wk_ref[...], preferred_element_type=jnp.float32)
    x1 = kp[:, :128]
    x2 = kp[:, 128:]
    vp = jnp.dot(feats, wv_ref[...], preferred_element_type=jnp.float32)
    kv_ref[...] = jnp.concatenate(
        [_pack_bf16(x1 * c - x2 * s, x1 * s + x2 * c),
         _pack_bf16(vp[:, :128], vp[:, 128:])], axis=-1)


def _out_body(o_ref, res_ref, w_ref, out_ref):
    out_ref[...] = jnp.dot(o_ref[...], w_ref[...],
                           preferred_element_type=jnp.float32) + res_ref[...]


_ROWS = 48
_UPQ = KPAD // _ROWS
_NSLOT = 8
_AHEAD = 4


@functools.partial(
    pl.kernel,
    out_type=jax.ShapeDtypeStruct((2048, KPAD, EMBED), jnp.int32),
    mesh=plsc.VectorSubcoreMesh(core_axis_name="c", subcore_axis_name="s"),
    scratch_types=[
        pltpu.VMEM((64, _UPQ, _ROWS), jnp.int32),
        pltpu.VMEM((_NSLOT, _ROWS, EMBED), jnp.int32),
        [pltpu.SemaphoreType.DMA] * _NSLOT,
        [pltpu.SemaphoreType.DMA] * _NSLOT,
    ],
)
def _sc_gather(kv_tab, idx3, gkv_hbm, idx_v, bufs, gsems, wsems):
    nc = 2
    wid = lax.axis_index("s") * nc + lax.axis_index("c")
    qpw = 2048 // 32
    nu = _UPQ * qpw
    ds = pl.ds

    pltpu.sync_copy(idx3.at[ds(wid * qpw, qpw)], idx_v)

    def issue_g(qi, quarter, b):
        pltpu.async_copy(kv_tab.at[idx_v.at[qi, quarter]], bufs.at[b], gsems[b])

    def wait_g(b):
        pltpu.make_async_copy(kv_tab.at[ds(0, _ROWS)], bufs.at[b], gsems[b]).wait()

    def issue_w(qi, quarter, b):
        q = wid * qpw + qi
        pltpu.async_copy(bufs.at[b], gkv_hbm.at[q, ds(quarter * _ROWS, _ROWS)],
                         wsems[b])

    def wait_w(b):
        pltpu.make_async_copy(bufs.at[b], gkv_hbm.at[0, ds(0, _ROWS)],
                              wsems[b]).wait()

    def qh(u_base, b):
        return u_base + b // _UPQ, b % _UPQ

    for u in range(_AHEAD):
        issue_g(u // _UPQ, u % _UPQ, u % _NSLOT)

    for b in range(_NSLOT):
        u = b
        wait_g(b)
        issue_w(u // _UPQ, u % _UPQ, b)
        u2 = u + _AHEAD
        if u2 >= _NSLOT:
            wait_w(u2 % _NSLOT)
        issue_g(u2 // _UPQ, u2 % _UPQ, u2 % _NSLOT)

    def body(m, _):
        qb = m * (_NSLOT // _UPQ)
        for b in range(_NSLOT):
            wait_g(b)
            qi, qq = qh(qb, b)
            issue_w(qi, qq, b)
            b2 = (b + _AHEAD) % _NSLOT
            wait_w(b2)
            qi2, qq2 = qh(qb, b + _AHEAD)
            issue_g(qi2, qq2, b2)
        return 0

    lax.fori_loop(1, nu // _NSLOT - 1, body, 0)

    qb_last = nu // _UPQ - _NSLOT // _UPQ
    for b in range(_NSLOT):
        u = nu - _NSLOT + b
        wait_g(b)
        issue_w(u // _UPQ, u % _UPQ, b)
        u2 = u + _AHEAD
        if u2 < nu:
            wait_w(u2 % _NSLOT)
            qi2, qq2 = qh(qb_last, b + _AHEAD)
            issue_g(qi2, qq2, u2 % _NSLOT)

    for b in range(_NSLOT):
        wait_w(b)


def _attn_body(gkv_ref, q_ref, bias_ref, o_ref):
    nb = q_ref.shape[0]
    ji = lax.broadcasted_iota(jnp.int32, (EMBED, HEADS), 0)
    hi = lax.broadcasted_iota(jnp.int32, (EMBED, HEADS), 1)
    hm = ((ji % 128) // HALF == hi).astype(jnp.bfloat16)
    qm = q_ref[...].astype(jnp.bfloat16)[:, :, None] * hm[None]
    gkp = gkv_ref[:, :, :128]
    gk = jnp.concatenate([_unpack_hi(gkp), _unpack_lo(gkp)],
                         axis=-1).astype(jnp.bfloat16)
    logits = lax.dot_general(qm, gk,
                             (((1,), (2,)), ((0,), (0,))),
                             preferred_element_type=jnp.float32)
    logits = logits + bias_ref[...][:, None, :]
    m = jnp.max(logits, axis=-1, keepdims=True)
    e = jnp.exp(logits - m)
    attn = (e / jnp.sum(e, axis=-1, keepdims=True)).astype(jnp.bfloat16)
    attnb = lax.dot_general(attn, hm,
                            (((1,), (1,)), ((), ())),
                            preferred_element_type=jnp.float32)
    gvp = gkv_ref[:, :, 128:]
    gv = jnp.concatenate([_unpack_hi(gvp), _unpack_lo(gvp)], axis=-1)
    o_ref[...] = jnp.sum(attnb * gv, axis=1)


def kernel(query, query_spatial_positions, query_batch_offsets, stacked_feature_maps,
           level_spatial_shapes, norm_w, norm_b, Wq, Wkv, Wout, rope_freqs):
    n = query.shape[0]
    perm = _PERM
    Wq_p = Wq[perm, :]
    Wk, Wv = jnp.split(Wkv, 2, axis=0)
    Wk_p = Wk[perm, :]
    Wv_p = Wv[perm, :]
    Wout_p = Wout[:, perm]
    f3 = rope_freqs.reshape(3, 128)

    shapes_f = level_spatial_shapes.astype(jnp.float32)
    max_shape = level_spatial_shapes.max(0)
    max_shape_f = max_shape.astype(jnp.float32)
    max_level = jnp.argmax(jnp.prod(level_spatial_shapes, -1)).astype(jnp.float32)
    lvterm = max_level * f3[2:3, :]
    sb = (max_shape_f / shapes_f)
    sb_row = jnp.concatenate([sb[:, 0], sb[:, 1]]).reshape(1, 2 * NLEV)

    q_rot = pl.pallas_call(
        _q_body,
        grid=(n // 256,),
        in_specs=[
            pl.BlockSpec((256, EMBED), lambda i: (i, 0)),
            pl.BlockSpec((256, 2), lambda i: (i, 0)),
            pl.BlockSpec((EMBED, EMBED), lambda i: (0, 0)),
            pl.BlockSpec((1, EMBED), lambda i: (0, 0)),
            pl.BlockSpec((1, EMBED), lambda i: (0, 0)),
            pl.BlockSpec((3, 128), lambda i: (0, 0)),
            pl.BlockSpec((1, 128), lambda i: (0, 0)),
        ],
        out_specs=pl.BlockSpec((256, EMBED), lambda i: (i, 0)),
        out_shape=jax.ShapeDtypeStruct((n, EMBED), jnp.float32),
    )(query, query_spatial_positions, Wq_p.T, norm_w.reshape(1, EMBED),
      norm_b.reshape(1, EMBED), f3, lvterm)

    S = stacked_feature_maps.reshape(-1, EMBED)
    T = S.shape[0]
    blk = 1024
    kv_tab = pl.pallas_call(
        functools.partial(_kv_body, blk=blk),
        grid=(T // blk,),
        in_specs=[
            pl.BlockSpec((blk, EMBED), lambda i: (i, 0)),
            pl.BlockSpec((EMBED, EMBED), lambda i: (0, 0)),
            pl.BlockSpec((EMBED, EMBED), lambda i: (0, 0)),
            pl.BlockSpec((3, 128), lambda i: (0, 0)),
            pl.BlockSpec((1, 2 * NLEV), lambda i: (0, 0)),
        ],
        out_specs=pl.BlockSpec((blk, EMBED), lambda i: (i, 0)),
        out_shape=jax.ShapeDtypeStruct((T, EMBED), jnp.int32),
    )(S, Wk_p.T, Wv_p.T, f3, sb_row)

    grids, lev_np = _build_offset_grids()
    lev_ids = jnp.asarray(lev_np)
    scal = shapes_f / max_shape_f
    parts = [jnp.floor(query_spatial_positions * scal[l]).astype(jnp.int32)[:, None, :]
             + jnp.asarray(grids[l])[None] for l in range(NLEV)]
    nh = jnp.concatenate(parts, 1)
    lshape_k = level_spatial_shapes[lev_ids]
    valid = jnp.all((nh >= 0) & (nh < lshape_k[None]), -1)
    yc = jnp.clip(nh[..., 0], 0, MAXHW - 1)
    xc = jnp.clip(nh[..., 1], 0, MAXHW - 1)
    bids = (jnp.arange(n, dtype=jnp.int32) >= query_batch_offsets[1]).astype(jnp.int32)
    flat = ((bids[:, None] * MAXHW + yc) * MAXHW + xc) * NLEV + lev_ids[None]
    flat_p = jnp.concatenate([flat, jnp.zeros((n, KPAD - NKEY), jnp.int32)], 1)
    bias = jnp.where(
        jnp.concatenate([valid, jnp.zeros((n, KPAD - NKEY), bool)], 1),
        0.0, -1e9).astype(jnp.float32)
    idx3 = flat_p.reshape(n, _UPQ, _ROWS)

    gkv = _sc_gather(kv_tab, idx3)

    nb = 32
    o = pl.pallas_call(
        _attn_body,
        grid=(n // nb,),
        in_specs=[
            pl.BlockSpec((nb, KPAD, EMBED), lambda i: (i, 0, 0)),
            pl.BlockSpec((nb, EMBED), lambda i: (i, 0)),
            pl.BlockSpec((nb, KPAD), lambda i: (i, 0)),
        ],
        out_specs=pl.BlockSpec((nb, EMBED), lambda i: (i, 0)),
        out_shape=jax.ShapeDtypeStruct((n, EMBED), jnp.float32),
    )(gkv, q_rot, bias)

    x = pl.pallas_call(
        _out_body,
        grid=(n // 256,),
        in_specs=[
            pl.BlockSpec((256, EMBED), lambda i: (i, 0)),
            pl.BlockSpec((256, EMBED), lambda i: (i, 0)),
            pl.BlockSpec((EMBED, EMBED), lambda i: (0, 0)),
        ],
        out_specs=pl.BlockSpec((256, EMBED), lambda i: (i, 0)),
        out_shape=jax.ShapeDtypeStruct((n, EMBED), jnp.float32),
    )(o, query, Wout_p.T)
    return x


# speedup vs baseline: 3.6546x; 3.6546x over previous
import functools

import jax
import jax.numpy as jnp
import numpy as np
from jax import lax
from jax.experimental import pallas as pl
from jax.experimental.pallas import tpu as pltpu
from jax.experimental.pallas import tpu_sc as plsc

EMBED = 256
HEADS = 8
HD = EMBED // HEADS
HALF = HD // 2
NLEV = 4
NH_SIZES = [3, 5, 7, 9]
NKEY = sum(s * s for s in NH_SIZES)
MAXHW = 96
LSIZES = [96, 48, 24, 12]
RADII = [1, 2, 3, 4]

TILE = 12
TPS = MAXHW // TILE
NT = 2 * TPS * TPS
QCAP = 64
NSLOT = NT * QCAP
WINW = [14, 10, 9, 10]
WINX = [24, 24, 16, 24]
KW = 1024
PAYW = 384

_PERM = np.concatenate([np.arange(0, EMBED, 2), np.arange(1, EMBED, 2)])
_HI_MASK = np.int32(np.uint32(0xFFFF0000).view(np.int32))


def _pack_bf16(x1, x2):
    b1 = lax.bitcast_convert_type(x1, jnp.int32)
    b2 = lax.bitcast_convert_type(x2, jnp.int32)
    r1 = b1 + 0x7FFF + (lax.shift_right_logical(b1, 16) & 1)
    r2 = b2 + 0x7FFF + (lax.shift_right_logical(b2, 16) & 1)
    return (r1 & _HI_MASK) | lax.shift_right_logical(r2, 16)


def _unpack_hi(p):
    return lax.bitcast_convert_type(p & _HI_MASK, jnp.float32)


def _unpack_lo(p):
    return lax.bitcast_convert_type(p << 16, jnp.float32)


def _q_body(q_ref, pos_ref, w_ref, nw_ref, nb_ref, f3_ref, lv_ref, out_ref):
    q = q_ref[...]
    mu = jnp.mean(q, axis=-1, keepdims=True)
    var = jnp.mean((q - mu) ** 2, axis=-1, keepdims=True)
    qn = (q - mu) * jax.lax.rsqrt(var + 1e-5) * nw_ref[...] + nb_ref[...]
    qp = jnp.dot(qn, w_ref[...], preferred_element_type=jnp.float32)
    ang = (pos_ref[:, 0:1] * f3_ref[0:1, :] + pos_ref[:, 1:2] * f3_ref[1:2, :]
           + lv_ref[...])
    c = jnp.cos(ang)
    s = jnp.sin(ang)
    x1 = qp[:, :128]
    x2 = qp[:, 128:]
    scale = 1.0 / np.sqrt(np.float32(HD))
    out_ref[...] = jnp.concatenate([(x1 * c - x2 * s) * scale,
                                    (x1 * s + x2 * c) * scale], axis=-1)


def _kv_body(s_ref, wk_ref, wv_ref, f3_ref, sb_ref, kv_ref, *, blk):
    i = pl.program_id(0)
    feats = s_ref[...]
    r = i * blk + lax.broadcasted_iota(jnp.int32, (blk, 1), 0)
    l_r = r // (2 * MAXHW * MAXHW)
    rem = r % (MAXHW * MAXHW)
    x_r = rem % MAXHW
    y_r = rem // MAXHW
    lf = l_r.astype(jnp.float32)
    sby = jnp.zeros((blk, 1), jnp.float32)
    sbx = jnp.zeros((blk, 1), jnp.float32)
    for l in range(NLEV):
        m = (l_r == l).astype(jnp.float32)
        sby = sby + m * sb_ref[0:1, l:l + 1]
        sbx = sbx + m * sb_ref[0:1, NLEV + l:NLEV + l + 1]
    py = (y_r.astype(jnp.float32) + 0.5) * sby - 0.5
    px = (x_r.astype(jnp.float32) + 0.5) * sbx - 0.5
    ang = py * f3_ref[0:1, :] + px * f3_ref[1:2, :] + lf * f3_ref[2:3, :]
    c = jnp.cos(ang)
    s = jnp.sin(ang)
    kp = jnp.dot(feats, wk_ref[...], preferred_element_type=jnp.float32)
    x1 = kp[:, :128]
    x2 = kp[:, 128:]
    vp = jnp.dot(feats, wv_ref[...], preferred_element_type=jnp.float32)
    kv_ref[...] = jnp.concatenate(
        [_pack_bf16(x1 * c - x2 * s, x1 * s + x2 * c),
         _pack_bf16(vp[:, :128], vp[:, 128:])], axis=-1)


def _out_body(o_ref, res_ref, w_ref, out_ref):
    out_ref[...] = jnp.dot(o_ref[...], w_ref[...],
                           preferred_element_type=jnp.float32) + res_ref[...]


def _make_sc_rowgather(tab_n, d, nrows, dtype):
    rpw = nrows // 32
    ch = 64
    nch = rpw // ch

    @functools.partial(
        pl.kernel,
        out_type=jax.ShapeDtypeStruct((nrows, d), dtype),
        mesh=plsc.VectorSubcoreMesh(core_axis_name="c", subcore_axis_name="s"),
        scratch_types=[
            pltpu.VMEM((rpw,), jnp.int32),
            pltpu.VMEM((ch, d), dtype),
            pltpu.SemaphoreType.DMA,
        ],
    )
    def gather(tab, idx, out_hbm, idx_v, buf, sem):
        wid = lax.axis_index("s") * 2 + lax.axis_index("c")
        base = wid * rpw
        pltpu.sync_copy(idx.at[pl.ds(base, rpw)], idx_v)
        for cc in range(nch):
            pltpu.async_copy(tab.at[idx_v.at[pl.ds(cc * ch, ch)]], buf,
                             sem).wait()
            pltpu.sync_copy(buf, out_hbm.at[pl.ds(base + cc * ch, ch)])

    return gather


_sc_perm_q = _make_sc_rowgather(2049, PAYW, NSLOT, jnp.float32)
_sc_unperm_o = _make_sc_rowgather(NSLOT, EMBED, 2048, jnp.float32)


def _win_starts(ty, tx):
    out = []
    ideal = [(ty * 12 - 1, tx * 12 - 1),
             (ty * 6 - 2, tx * 6 - 2),
             (ty * 3 - 3, tx * 3 - 3),
             ((ty * 3) // 2 - 4, (tx * 3) // 2 - 4)]
    for l in range(NLEV):
        s = LSIZES[l]
        ys = jnp.clip(ideal[l][0], 0, s - WINW[l])
        xc = jnp.clip(ideal[l][1], 0, max(s - WINW[l], 0))
        xs = (jnp.minimum(xc, MAXHW - WINX[l]) // 8) * 8
        out.append((ys, xs))
    return out


def _attn_compute(q_ref, wins, o_ref, starts):
    qb = q_ref[...]
    q = qb[:, :EMBED]
    qy = qb[:, EMBED:EMBED + 1]
    qx = qb[:, EMBED + 1:EMBED + 2]

    ks = []
    vs = []
    biases = []
    for l in range(NLEV):
        wy, wx = WINW[l], WINX[l]
        win = wins[l][...].reshape(wy * wx, EMBED)
        kp = win[:, :128]
        vp = win[:, 128:]
        ks.append(jnp.concatenate([_unpack_hi(kp), _unpack_lo(kp)], -1))
        vs.append(jnp.concatenate([_unpack_hi(vp), _unpack_lo(vp)], -1))
        ys, xs = starts[l]
        j = lax.broadcasted_iota(jnp.int32, (QCAP, wy * wx), 1)
        ky = ys + j // wx
        kx = xs + j % wx
        sc = np.float32(LSIZES[l] / MAXHW)
        lpy = jnp.floor(qy * sc).astype(jnp.int32)
        lpx = jnp.floor(qx * sc).astype(jnp.int32)
        ok = ((jnp.abs(ky - lpy) <= RADII[l]) &
              (jnp.abs(kx - lpx) <= RADII[l]) &
              (kx < LSIZES[l]))
        biases.append(jnp.where(ok, 0.0, -1e9).astype(jnp.float32))
    npad = KW - sum(WINW[l] * WINX[l] for l in range(NLEV))
    K = jnp.concatenate(ks + [jnp.zeros((npad, EMBED), jnp.float32)], 0)
    V = jnp.concatenate(vs + [jnp.zeros((npad, EMBED), jnp.float32)], 0)
    bias = jnp.concatenate(
        biases + [jnp.full((QCAP, npad), -1e9, jnp.float32)], 1)
    Kb = K.astype(jnp.bfloat16)
    Vb = V.astype(jnp.bfloat16)

    o = jnp.zeros((QCAP, EMBED), jnp.float32)
    for h in range(HEADS):
        qh = jnp.concatenate([q[:, 16 * h:16 * h + 16],
                              q[:, 128 + 16 * h:128 + 16 * h + 16]], -1)
        kh = jnp.concatenate([Kb[:, 16 * h:16 * h + 16],
                              Kb[:, 128 + 16 * h:128 + 16 * h + 16]], -1)
        logits = lax.dot_general(qh.astype(jnp.bfloat16), kh,
                                 (((1,), (1,)), ((), ())),
                                 preferred_element_type=jnp.float32)
        logits = logits + bias
        mx = jnp.max(logits, axis=-1, keepdims=True)
        e = jnp.exp(logits - mx)
        attn = (e / jnp.sum(e, axis=-1, keepdims=True)).astype(jnp.bfloat16)
        ov = lax.dot_general(attn, Vb, (((1,), (0,)), ((), ())),
                             preferred_element_type=jnp.float32)
        ji = lax.broadcasted_iota(jnp.int32, (1, EMBED), 1)
        hsel = ((ji % 128) // HALF == h).astype(jnp.float32)
        o = o + ov * hsel
    o_ref[...] = o


def _attn_body(kvt_ref, q_ref, o_ref, w0a, w1a, w2a, w3a, w0b, w1b, w2b, w3b,
               sema, semb):
    i = pl.program_id(0)
    winsa = [w0a, w1a, w2a, w3a]
    winsb = [w0b, w1b, w2b, w3b]

    def tile_of(t):
        b = t // (TPS * TPS)
        rem = t % (TPS * TPS)
        return b, rem // TPS, rem % TPS

    def issue(t, wins, sem):
        b, ty, tx = tile_of(t)
        starts = _win_starts(ty, tx)
        for l in range(NLEV):
            ys, xs = starts[l]
            pltpu.make_async_copy(
                kvt_ref.at[l, b, pl.ds(ys, WINW[l]), pl.ds(xs, WINX[l])],
                wins[l], sem).start()

    def wait(wins, sem):
        for l in range(NLEV):
            pltpu.make_async_copy(
                kvt_ref.at[l, 0, pl.ds(0, WINW[l]), pl.ds(0, WINX[l])],
                wins[l], sem).wait()

    @pl.when(i == 0)
    def _():
        issue(0, winsa, sema)

    @pl.when((i + 1 < NT) & ((i + 1) % 2 == 1))
    def _():
        issue(i + 1, winsb, semb)

    @pl.when((i + 1 < NT) & ((i + 1) % 2 == 0))
    def _():
        issue(i + 1, winsa, sema)

    _, ty, tx = tile_of(i)
    starts = _win_starts(ty, tx)

    @pl.when(i % 2 == 0)
    def _():
        wait(winsa, sema)
        _attn_compute(q_ref, winsa, o_ref, starts)

    @pl.when(i % 2 == 1)
    def _():
        wait(winsb, semb)
        _attn_compute(q_ref, winsb, o_ref, starts)


def kernel(query, query_spatial_positions, query_batch_offsets, stacked_feature_maps,
           level_spatial_shapes, norm_w, norm_b, Wq, Wkv, Wout, rope_freqs):
    n = query.shape[0]
    perm = _PERM
    Wq_p = Wq[perm, :]
    Wk, Wv = jnp.split(Wkv, 2, axis=0)
    Wk_p = Wk[perm, :]
    Wv_p = Wv[perm, :]
    Wout_p = Wout[:, perm]
    f3 = rope_freqs.reshape(3, 128)

    shapes_f = level_spatial_shapes.astype(jnp.float32)
    max_shape = level_spatial_shapes.max(0)
    max_shape_f = max_shape.astype(jnp.float32)
    max_level = jnp.argmax(jnp.prod(level_spatial_shapes, -1)).astype(jnp.float32)
    lvterm = max_level * f3[2:3, :]
    sb = (max_shape_f / shapes_f)
    sb_row = jnp.concatenate([sb[:, 0], sb[:, 1]]).reshape(1, 2 * NLEV)

    q_rot = pl.pallas_call(
        _q_body,
        grid=(n // 256,),
        in_specs=[
            pl.BlockSpec((256, EMBED), lambda i: (i, 0)),
            pl.BlockSpec((256, 2), lambda i: (i, 0)),
            pl.BlockSpec((EMBED, EMBED), lambda i: (0, 0)),
            pl.BlockSpec((1, EMBED), lambda i: (0, 0)),
            pl.BlockSpec((1, EMBED), lambda i: (0, 0)),
            pl.BlockSpec((3, 128), lambda i: (0, 0)),
            pl.BlockSpec((1, 128), lambda i: (0, 0)),
        ],
        out_specs=pl.BlockSpec((256, EMBED), lambda i: (i, 0)),
        out_shape=jax.ShapeDtypeStruct((n, EMBED), jnp.float32),
    )(query, query_spatial_positions, Wq_p.T, norm_w.reshape(1, EMBED),
      norm_b.reshape(1, EMBED), f3, lvterm)

    S = stacked_feature_maps.transpose(3, 0, 1, 2, 4).reshape(-1, EMBED)
    T = S.shape[0]
    blk = 1024
    kv_tab = pl.pallas_call(
        functools.partial(_kv_body, blk=blk),
        grid=(T // blk,),
        in_specs=[
            pl.BlockSpec((blk, EMBED), lambda i: (i, 0)),
            pl.BlockSpec((EMBED, EMBED), lambda i: (0, 0)),
            pl.BlockSpec((EMBED, EMBED), lambda i: (0, 0)),
            pl.BlockSpec((3, 128), lambda i: (0, 0)),
            pl.BlockSpec((1, 2 * NLEV), lambda i: (0, 0)),
        ],
        out_specs=pl.BlockSpec((blk, EMBED), lambda i: (i, 0)),
        out_shape=jax.ShapeDtypeStruct((T, EMBED), jnp.int32),
    )(S, Wk_p.T, Wv_p.T, f3, sb_row)
    kvt = kv_tab.reshape(NLEV, 2, MAXHW, MAXHW, EMBED)

    bids = (jnp.arange(n, dtype=jnp.int32) >= query_batch_offsets[1]).astype(jnp.int32)
    qy = query_spatial_positions[:, 0]
    qx = query_spatial_positions[:, 1]
    ty = jnp.clip(jnp.floor(qy / TILE).astype(jnp.int32), 0, TPS - 1)
    tx = jnp.clip(jnp.floor(qx / TILE).astype(jnp.int32), 0, TPS - 1)
    tid = bids * (TPS * TPS) + ty * TPS + tx
    order = jnp.argsort(tid, stable=True).astype(jnp.int32)
    tid_sorted = tid[order]
    rank = jnp.arange(n, dtype=jnp.int32) - jnp.searchsorted(
        tid_sorted, tid_sorted, side='left').astype(jnp.int32)
    slot_sorted = tid_sorted * QCAP + jnp.minimum(rank, QCAP - 1)
    perm_slot = jnp.full((NSLOT,), n, jnp.int32).at[slot_sorted].set(order)
    slot_q = jnp.zeros((n,), jnp.int32).at[order].set(slot_sorted)

    payload = jnp.concatenate(
        [q_rot, query_spatial_positions,
         jnp.zeros((n, PAYW - EMBED - 2), jnp.float32)], 1)
    payload = jnp.concatenate([payload, jnp.zeros((1, PAYW), jnp.float32)], 0)

    qperm = _sc_perm_q(payload, perm_slot)

    o_slots = pl.pallas_call(
        _attn_body,
        grid=(NT,),
        in_specs=[
            pl.BlockSpec(memory_space=pl.ANY),
            pl.BlockSpec((QCAP, PAYW), lambda i: (i, 0)),
        ],
        out_specs=pl.BlockSpec((QCAP, EMBED), lambda i: (i, 0)),
        out_shape=jax.ShapeDtypeStruct((NSLOT, EMBED), jnp.float32),
        scratch_shapes=(
            [pltpu.VMEM((WINW[l], WINX[l], EMBED), jnp.int32) for l in range(NLEV)]
            + [pltpu.VMEM((WINW[l], WINX[l], EMBED), jnp.int32) for l in range(NLEV)]
            + [pltpu.SemaphoreType.DMA, pltpu.SemaphoreType.DMA]),
    )(kvt, qperm)

    o = _sc_unperm_o(o_slots, slot_q)

    x = pl.pallas_call(
        _out_body,
        grid=(n // 256,),
        in_specs=[
            pl.BlockSpec((256, EMBED), lambda i: (i, 0)),
            pl.BlockSpec((256, EMBED), lambda i: (i, 0)),
            pl.BlockSpec((EMBED, EMBED), lambda i: (0, 0)),
        ],
        out_specs=pl.BlockSpec((256, EMBED), lambda i: (i, 0)),
        out_shape=jax.ShapeDtypeStruct((n, EMBED), jnp.float32),
    )(o, query, Wout_p.T)
    return x


# speedup vs baseline: 3.7581x; 1.0283x over previous
import functools

import jax
import jax.numpy as jnp
import numpy as np
from jax import lax
from jax.experimental import pallas as pl
from jax.experimental.pallas import tpu as pltpu
from jax.experimental.pallas import tpu_sc as plsc

EMBED = 256
HEADS = 8
HD = EMBED // HEADS
HALF = HD // 2
NLEV = 4
NH_SIZES = [3, 5, 7, 9]
NKEY = sum(s * s for s in NH_SIZES)
MAXHW = 96
LSIZES = [96, 48, 24, 12]
RADII = [1, 2, 3, 4]

TILE = 12
TPS = MAXHW // TILE
NT = 2 * TPS * TPS
QCAP = 48
NSLOT = NT * QCAP
WINW = [14, 10, 9, 10]
WINX = [24, 24, 16, 24]
KW = 1024

_PERM = np.concatenate([np.arange(0, EMBED, 2), np.arange(1, EMBED, 2)])
_HI_MASK = np.int32(np.uint32(0xFFFF0000).view(np.int32))


def _pack_bf16(x1, x2):
    b1 = lax.bitcast_convert_type(x1, jnp.int32)
    b2 = lax.bitcast_convert_type(x2, jnp.int32)
    r1 = b1 + 0x7FFF + (lax.shift_right_logical(b1, 16) & 1)
    r2 = b2 + 0x7FFF + (lax.shift_right_logical(b2, 16) & 1)
    return (r1 & _HI_MASK) | lax.shift_right_logical(r2, 16)


def _unpack_hi(p):
    return lax.bitcast_convert_type(p & _HI_MASK, jnp.float32)


def _unpack_lo(p):
    return lax.bitcast_convert_type(p << 16, jnp.float32)


def _q_body(q_ref, pos_ref, w_ref, nw_ref, nb_ref, f3_ref, lv_ref, out_ref):
    q = q_ref[...]
    mu = jnp.mean(q, axis=-1, keepdims=True)
    var = jnp.mean((q - mu) ** 2, axis=-1, keepdims=True)
    qn = (q - mu) * jax.lax.rsqrt(var + 1e-5) * nw_ref[...] + nb_ref[...]
    qp = jnp.dot(qn, w_ref[...], preferred_element_type=jnp.float32)
    ang = (pos_ref[:, 0:1] * f3_ref[0:1, :] + pos_ref[:, 1:2] * f3_ref[1:2, :]
           + lv_ref[...])
    c = jnp.cos(ang)
    s = jnp.sin(ang)
    x1 = qp[:, :128]
    x2 = qp[:, 128:]
    scale = 1.0 / np.sqrt(np.float32(HD))
    out_ref[...] = jnp.concatenate([(x1 * c - x2 * s) * scale,
                                    (x1 * s + x2 * c) * scale], axis=-1)


def _kv_body(s_ref, wk_ref, wv_ref, f3_ref, sb_ref, kv_ref, *, blk):
    i = pl.program_id(0)
    feats = s_ref[...]
    r = i * blk + lax.broadcasted_iota(jnp.int32, (blk, 1), 0)
    l_r = r // (2 * MAXHW * MAXHW)
    rem = r % (MAXHW * MAXHW)
    x_r = rem % MAXHW
    y_r = rem // MAXHW
    lf = l_r.astype(jnp.float32)
    sby = jnp.zeros((blk, 1), jnp.float32)
    sbx = jnp.zeros((blk, 1), jnp.float32)
    for l in range(NLEV):
        m = (l_r == l).astype(jnp.float32)
        sby = sby + m * sb_ref[0:1, l:l + 1]
        sbx = sbx + m * sb_ref[0:1, NLEV + l:NLEV + l + 1]
    py = (y_r.astype(jnp.float32) + 0.5) * sby - 0.5
    px = (x_r.astype(jnp.float32) + 0.5) * sbx - 0.5
    ang = py * f3_ref[0:1, :] + px * f3_ref[1:2, :] + lf * f3_ref[2:3, :]
    c = jnp.cos(ang)
    s = jnp.sin(ang)
    kp = jnp.dot(feats, wk_ref[...], preferred_element_type=jnp.float32)
    x1 = kp[:, :128]
    x2 = kp[:, 128:]
    vp = jnp.dot(feats, wv_ref[...], preferred_element_type=jnp.float32)
    kv_ref[...] = jnp.concatenate(
        [_pack_bf16(x1 * c - x2 * s, x1 * s + x2 * c),
         _pack_bf16(vp[:, :128], vp[:, 128:])], axis=-1)


def _out_body(o_ref, res_ref, w_ref, out_ref):
    out_ref[...] = jnp.dot(o_ref[...], w_ref[...],
                           preferred_element_type=jnp.float32) + res_ref[...]


def _make_sc_rowgather(tab_n, d, nrows, dtype):
    rpw = nrows // 32
    ch = 64
    nch = rpw // ch

    @functools.partial(
        pl.kernel,
        out_type=jax.ShapeDtypeStruct((nrows, d), dtype),
        mesh=plsc.VectorSubcoreMesh(core_axis_name="c", subcore_axis_name="s"),
        scratch_types=[
            pltpu.VMEM((rpw,), jnp.int32),
            pltpu.VMEM((ch, d), dtype),
            pltpu.SemaphoreType.DMA,
        ],
    )
    def gather(tab, idx, out_hbm, idx_v, buf, sem):
        wid = lax.axis_index("s") * 2 + lax.axis_index("c")
        base = wid * rpw
        pltpu.sync_copy(idx.at[pl.ds(base, rpw)], idx_v)
        for cc in range(nch):
            pltpu.async_copy(tab.at[idx_v.at[pl.ds(cc * ch, ch)]], buf,
                             sem).wait()
            pltpu.sync_copy(buf, out_hbm.at[pl.ds(base + cc * ch, ch)])

    return gather


_sc_perm_q = _make_sc_rowgather(2049, EMBED, NSLOT, jnp.float32)
_sc_unperm_o = _make_sc_rowgather(NSLOT, EMBED, 2048, jnp.float32)


def _win_starts(ty, tx):
    out = []
    ideal = [(ty * 12 - 1, tx * 12 - 1),
             (ty * 6 - 2, tx * 6 - 2),
             (ty * 3 - 3, tx * 3 - 3),
             ((ty * 3) // 2 - 4, (tx * 3) // 2 - 4)]
    for l in range(NLEV):
        s = LSIZES[l]
        ys = jnp.clip(ideal[l][0], 0, s - WINW[l])
        xc = jnp.clip(ideal[l][1], 0, max(s - WINW[l], 0))
        xs = (jnp.minimum(xc, MAXHW - WINX[l]) // 8) * 8
        out.append((ys, xs))
    return out


def _attn_compute(q_ref, qp_ref, wins, o_ref, starts):
    q = q_ref[...]
    qy = qp_ref[:, 0:1]
    qx = qp_ref[:, 1:2]

    ks = []
    vs = []
    biases = []
    for l in range(NLEV):
        wy, wx = WINW[l], WINX[l]
        win = wins[l][...].reshape(wy * wx, EMBED)
        kp = win[:, :128]
        vp = win[:, 128:]
        ks.append(jnp.concatenate([_unpack_hi(kp), _unpack_lo(kp)], -1))
        vs.append(jnp.concatenate([_unpack_hi(vp), _unpack_lo(vp)], -1))
        ys, xs = starts[l]
        j = lax.broadcasted_iota(jnp.int32, (QCAP, wy * wx), 1)
        ky = ys + j // wx
        kx = xs + j % wx
        sc = np.float32(LSIZES[l] / MAXHW)
        lpy = jnp.floor(qy * sc).astype(jnp.int32)
        lpx = jnp.floor(qx * sc).astype(jnp.int32)
        ok = ((jnp.abs(ky - lpy) <= RADII[l]) &
              (jnp.abs(kx - lpx) <= RADII[l]) &
              (kx < LSIZES[l]))
        biases.append(jnp.where(ok, 0.0, -1e9).astype(jnp.float32))
    npad = KW - sum(WINW[l] * WINX[l] for l in range(NLEV))
    K = jnp.concatenate(ks + [jnp.zeros((npad, EMBED), jnp.float32)], 0)
    V = jnp.concatenate(vs + [jnp.zeros((npad, EMBED), jnp.float32)], 0)
    bias = jnp.concatenate(
        biases + [jnp.full((QCAP, npad), -1e9, jnp.float32)], 1)
    Kb = K.astype(jnp.bfloat16)
    Vb = V.astype(jnp.bfloat16)

    o = jnp.zeros((QCAP, EMBED), jnp.float32)
    for h in range(HEADS):
        qh = jnp.concatenate([q[:, 16 * h:16 * h + 16],
                              q[:, 128 + 16 * h:128 + 16 * h + 16]], -1)
        kh = jnp.concatenate([Kb[:, 16 * h:16 * h + 16],
                              Kb[:, 128 + 16 * h:128 + 16 * h + 16]], -1)
        logits = lax.dot_general(qh.astype(jnp.bfloat16), kh,
                                 (((1,), (1,)), ((), ())),
                                 preferred_element_type=jnp.float32)
        logits = logits + bias
        mx = jnp.max(logits, axis=-1, keepdims=True)
        e = jnp.exp(logits - mx)
        attn = (e / jnp.sum(e, axis=-1, keepdims=True)).astype(jnp.bfloat16)
        ov = lax.dot_general(attn, Vb, (((1,), (0,)), ((), ())),
                             preferred_element_type=jnp.float32)
        ji = lax.broadcasted_iota(jnp.int32, (1, EMBED), 1)
        hsel = ((ji % 128) // HALF == h).astype(jnp.float32)
        o = o + ov * hsel
    o_ref[...] = o


def _attn_body(kvt_ref, q_ref, qp_ref, o_ref, w0a, w1a, w2a, w3a,
               w0b, w1b, w2b, w3b, sema, semb):
    i = pl.program_id(0)
    winsa = [w0a, w1a, w2a, w3a]
    winsb = [w0b, w1b, w2b, w3b]

    def tile_of(t):
        b = t // (TPS * TPS)
        rem = t % (TPS * TPS)
        return b, rem // TPS, rem % TPS

    def issue(t, wins, sem):
        b, ty, tx = tile_of(t)
        starts = _win_starts(ty, tx)
        for l in range(NLEV):
            ys, xs = starts[l]
            pltpu.make_async_copy(
                kvt_ref.at[l, b, pl.ds(ys, WINW[l]), pl.ds(xs, WINX[l])],
                wins[l], sem).start()

    def wait(wins, sem):
        for l in range(NLEV):
            pltpu.make_async_copy(
                kvt_ref.at[l, 0, pl.ds(0, WINW[l]), pl.ds(0, WINX[l])],
                wins[l], sem).wait()

    @pl.when(i == 0)
    def _():
        issue(0, winsa, sema)

    @pl.when((i + 1 < NT) & ((i + 1) % 2 == 1))
    def _():
        issue(i + 1, winsb, semb)

    @pl.when((i + 1 < NT) & ((i + 1) % 2 == 0))
    def _():
        issue(i + 1, winsa, sema)

    _, ty, tx = tile_of(i)
    starts = _win_starts(ty, tx)

    @pl.when(i % 2 == 0)
    def _():
        wait(winsa, sema)
        _attn_compute(q_ref, qp_ref, winsa, o_ref, starts)

    @pl.when(i % 2 == 1)
    def _():
        wait(winsb, semb)
        _attn_compute(q_ref, qp_ref, winsb, o_ref, starts)


def kernel(query, query_spatial_positions, query_batch_offsets, stacked_feature_maps,
           level_spatial_shapes, norm_w, norm_b, Wq, Wkv, Wout, rope_freqs):
    n = query.shape[0]
    perm = _PERM
    Wq_p = Wq[perm, :]
    Wk, Wv = jnp.split(Wkv, 2, axis=0)
    Wk_p = Wk[perm, :]
    Wv_p = Wv[perm, :]
    Wout_p = Wout[:, perm]
    f3 = rope_freqs.reshape(3, 128)

    shapes_f = level_spatial_shapes.astype(jnp.float32)
    max_shape = level_spatial_shapes.max(0)
    max_shape_f = max_shape.astype(jnp.float32)
    max_level = jnp.argmax(jnp.prod(level_spatial_shapes, -1)).astype(jnp.float32)
    lvterm = max_level * f3[2:3, :]
    sb = (max_shape_f / shapes_f)
    sb_row = jnp.concatenate([sb[:, 0], sb[:, 1]]).reshape(1, 2 * NLEV)

    q_rot = pl.pallas_call(
        _q_body,
        grid=(n // 256,),
        in_specs=[
            pl.BlockSpec((256, EMBED), lambda i: (i, 0)),
            pl.BlockSpec((256, 2), lambda i: (i, 0)),
            pl.BlockSpec((EMBED, EMBED), lambda i: (0, 0)),
            pl.BlockSpec((1, EMBED), lambda i: (0, 0)),
            pl.BlockSpec((1, EMBED), lambda i: (0, 0)),
            pl.BlockSpec((3, 128), lambda i: (0, 0)),
            pl.BlockSpec((1, 128), lambda i: (0, 0)),
        ],
        out_specs=pl.BlockSpec((256, EMBED), lambda i: (i, 0)),
        out_shape=jax.ShapeDtypeStruct((n, EMBED), jnp.float32),
    )(query, query_spatial_positions, Wq_p.T, norm_w.reshape(1, EMBED),
      norm_b.reshape(1, EMBED), f3, lvterm)

    S = stacked_feature_maps.transpose(3, 0, 1, 2, 4).reshape(-1, EMBED)
    T = S.shape[0]
    blk = 1024
    kv_tab = pl.pallas_call(
        functools.partial(_kv_body, blk=blk),
        grid=(T // blk,),
        in_specs=[
            pl.BlockSpec((blk, EMBED), lambda i: (i, 0)),
            pl.BlockSpec((EMBED, EMBED), lambda i: (0, 0)),
            pl.BlockSpec((EMBED, EMBED), lambda i: (0, 0)),
            pl.BlockSpec((3, 128), lambda i: (0, 0)),
            pl.BlockSpec((1, 2 * NLEV), lambda i: (0, 0)),
        ],
        out_specs=pl.BlockSpec((blk, EMBED), lambda i: (i, 0)),
        out_shape=jax.ShapeDtypeStruct((T, EMBED), jnp.int32),
    )(S, Wk_p.T, Wv_p.T, f3, sb_row)
    kvt = kv_tab.reshape(NLEV, 2, MAXHW, MAXHW, EMBED)

    bids = (jnp.arange(n, dtype=jnp.int32) >= query_batch_offsets[1]).astype(jnp.int32)
    qy = query_spatial_positions[:, 0]
    qx = query_spatial_positions[:, 1]
    ty = jnp.clip(jnp.floor(qy / TILE).astype(jnp.int32), 0, TPS - 1)
    tx = jnp.clip(jnp.floor(qx / TILE).astype(jnp.int32), 0, TPS - 1)
    tid = bids * (TPS * TPS) + ty * TPS + tx
    order = jnp.argsort(tid, stable=True).astype(jnp.int32)
    tid_sorted = tid[order]
    rank = jnp.arange(n, dtype=jnp.int32) - jnp.searchsorted(
        tid_sorted, tid_sorted, side='left').astype(jnp.int32)
    slot_sorted = tid_sorted * QCAP + jnp.minimum(rank, QCAP - 1)
    perm_slot = jnp.full((NSLOT,), n, jnp.int32).at[slot_sorted].set(order)
    slot_q = jnp.zeros((n,), jnp.int32).at[order].set(slot_sorted)

    payload = jnp.concatenate([q_rot, jnp.zeros((1, EMBED), jnp.float32)], 0)

    qperm = _sc_perm_q(payload, perm_slot)
    qpos_pad = jnp.concatenate(
        [query_spatial_positions, jnp.zeros((n, 6), jnp.float32)], 1)
    qpos_pad = jnp.concatenate([qpos_pad, jnp.zeros((1, 8), jnp.float32)], 0)
    qpos_perm = qpos_pad[perm_slot]

    o_slots = pl.pallas_call(
        _attn_body,
        grid=(NT,),
        in_specs=[
            pl.BlockSpec(memory_space=pl.ANY),
            pl.BlockSpec((QCAP, EMBED), lambda i: (i, 0)),
            pl.BlockSpec((QCAP, 8), lambda i: (i, 0)),
        ],
        out_specs=pl.BlockSpec((QCAP, EMBED), lambda i: (i, 0)),
        out_shape=jax.ShapeDtypeStruct((NSLOT, EMBED), jnp.float32),
        scratch_shapes=(
            [pltpu.VMEM((WINW[l], WINX[l], EMBED), jnp.int32) for l in range(NLEV)]
            + [pltpu.VMEM((WINW[l], WINX[l], EMBED), jnp.int32) for l in range(NLEV)]
            + [pltpu.SemaphoreType.DMA, pltpu.SemaphoreType.DMA]),
    )(kvt, qperm, qpos_perm)

    o = _sc_unperm_o(o_slots, slot_q)

    x = pl.pallas_call(
        _out_body,
        grid=(n // 256,),
        in_specs=[
            pl.BlockSpec((256, EMBED), lambda i: (i, 0)),
            pl.BlockSpec((256, EMBED), lambda i: (i, 0)),
            pl.BlockSpec((EMBED, EMBED), lambda i: (0, 0)),
        ],
        out_specs=pl.BlockSpec((256, EMBED), lambda i: (i, 0)),
        out_shape=jax.ShapeDtypeStruct((n, EMBED), jnp.float32),
    )(o, query, Wout_p.T)
    return x
